# baseline (speedup 1.0000x reference)
import functools

import jax
import jax.numpy as jnp
from jax import lax
from jax.experimental import pallas as pl
from jax.experimental.pallas import tpu as pltpu
from jax.experimental.pallas import tpu_sc as plsc

B_TOK = 4096
T_TOK = 200
EMB = 64
PEMB = 128
NW = 32
B = B_TOK * T_TOK
B_PER_W = B // NW
CH = 128
NCH = B_PER_W // CH

_NC = 2
NBUF = 4
_MESH = plsc.VectorSubcoreMesh(core_axis_name="c", subcore_axis_name="s")


@functools.partial(
    pl.kernel,
    mesh=_MESH,
    out_type=jax.ShapeDtypeStruct((B, PEMB), jnp.float32),
    scratch_types=[
        pltpu.VMEM((NCH, CH), jnp.int32),
        pltpu.VMEM((NBUF, CH, PEMB), jnp.float32),
        [pltpu.SemaphoreType.DMA] * NBUF,
        [pltpu.SemaphoreType.DMA] * NBUF,
    ],
    compiler_params=pltpu.CompilerParams(use_tc_tiling_on_sc=True),
)
def _gather_kernel(tok_hbm, table_hbm, out_hbm, idx_v, rows_v, gsems, wsems):
    wid = lax.axis_index("s") * _NC + lax.axis_index("c")
    base = wid * B_PER_W
    pltpu.sync_copy(tok_hbm.at[wid], idx_v)

    def gather(j, b):
        pltpu.async_copy(table_hbm.at[idx_v.at[j]], rows_v.at[b], gsems[b])

    def write(j, b):
        pltpu.async_copy(rows_v.at[b], out_hbm.at[pl.ds(base + j * CH, CH)],
                         wsems[b])

    def wait_gather(j, b):
        pltpu.make_async_copy(table_hbm.at[idx_v.at[b]], rows_v.at[b],
                              gsems[b]).wait()

    def wait_write(j, b):
        pltpu.make_async_copy(rows_v.at[b],
                              out_hbm.at[pl.ds(base + j * CH, CH)],
                              wsems[b]).wait()

    for b in range(NBUF):
        gather(b, b)

    def group(g, carry):
        for b in range(NBUF):
            j = g * NBUF + b
            wait_gather(j, b)
            write(j, b)
        for b in range(NBUF):
            j = g * NBUF + b
            wait_write(j, b)
            gather(j + NBUF, b)
        return carry

    lax.fori_loop(0, NCH // NBUF - 1, group, 0)

    for b in range(NBUF):
        j = NCH - NBUF + b
        wait_gather(j, b)
        write(j, b)
    for b in range(NBUF):
        j = NCH - NBUF + b
        wait_write(j, b)


VOCAB = 1000000
NFULL = VOCAB // CH
TAIL = VOCAB - NFULL * CH


@functools.partial(
    pl.kernel,
    mesh=_MESH,
    out_type=jax.ShapeDtypeStruct((VOCAB, PEMB), jnp.float32),
    scratch_types=[
        pltpu.VMEM((2, EMB, CH), jnp.float32),
        pltpu.VMEM((2, CH, PEMB), jnp.float32),
        [pltpu.SemaphoreType.DMA] * 2,
        [pltpu.SemaphoreType.DMA] * 2,
    ],
    compiler_params=pltpu.CompilerParams(use_tc_tiling_on_sc=True,
                                         needs_layout_passes=False),
)
def _fmt_kernel(tabt_hbm, tail_hbm, out_hbm, in_v, out_v, isems, osems):
    wid = lax.axis_index("s") * _NC + lax.axis_index("c")

    lane = lax.iota(jnp.int32, 16)
    NK = NFULL // NW

    def load(k, b):
        pltpu.async_copy(tabt_hbm.at[:, pl.ds((wid + k * NW) * CH, CH)],
                         in_v.at[b], isems[b])

    def wait_load(k, b):
        pltpu.make_async_copy(tabt_hbm.at[:, pl.ds((wid + k * NW) * CH, CH)],
                              in_v.at[b], isems[b]).wait()

    def store(k, b):
        pltpu.async_copy(out_v.at[b], out_hbm.at[pl.ds((wid + k * NW) * CH, CH)],
                         osems[b])

    def wait_store(k, b):
        pltpu.make_async_copy(out_v.at[b],
                              out_hbm.at[pl.ds((wid + k * NW) * CH, CH)],
                              osems[b]).wait()

    def transpose_block(b):
        def row(v, carry):
            col = jnp.full((16,), 0, jnp.int32) + v
            for kk in range(EMB // 16):
                g = plsc.load_gather(in_v.at[b], [lane + 16 * kk, col])
                out_v[b, v, pl.ds(16 * kk, 16)] = g
            return carry
        lax.fori_loop(0, CH, row, 0)

    load(0, 0)
    load(1, 1)

    def group(g, carry):
        for b in range(2):
            k = 2 * g + b
            wait_load(k, b)
            transpose_block(b)
            store(k, b)
        for b in range(2):
            k = 2 * g + b
            wait_store(k, b)

            @pl.when(k + 2 < NK)
            def _():
                load(k + 2, b)

        return carry

    lax.fori_loop(0, NK // 2, group, 0)

    @pl.when(wid < NFULL - NK * NW)
    def _():
        pltpu.sync_copy(tabt_hbm.at[:, pl.ds((wid + NK * NW) * CH, CH)],
                        in_v.at[0])
        transpose_block(0)
        pltpu.sync_copy(out_v.at[0],
                        out_hbm.at[pl.ds((wid + NK * NW) * CH, CH)])

    @pl.when(wid == NW - 1)
    def _():
        pltpu.sync_copy(tail_hbm, out_v.at[1, pl.ds(0, TAIL)])
        pltpu.sync_copy(out_v.at[1, pl.ds(0, TAIL)],
                        out_hbm.at[pl.ds(NFULL * CH, TAIL)])


def kernel(tokens, table):
    tok = tokens.reshape(NW, NCH, CH)
    tail = jnp.pad(table[NFULL * CH:], ((0, 0), (0, PEMB - EMB)))
    tab = _fmt_kernel(table.T, tail)
    out = _gather_kernel(tok, tab)
    return out[:, :EMB].reshape(B_TOK, T_TOK, EMB)


# speedup vs baseline: 1.0021x; 1.0021x over previous
import functools

import jax
import jax.numpy as jnp
from jax import lax
from jax.experimental import pallas as pl
from jax.experimental.pallas import tpu as pltpu
from jax.experimental.pallas import tpu_sc as plsc

B_TOK = 4096
T_TOK = 200
EMB = 64
PEMB = 128
NW = 32
B = B_TOK * T_TOK
B_PER_W = B // NW
CH = 128
NCH = B_PER_W // CH

_NC = 2
NBUF = 4
_MESH = plsc.VectorSubcoreMesh(core_axis_name="c", subcore_axis_name="s")


@functools.partial(
    pl.kernel,
    mesh=_MESH,
    out_type=jax.ShapeDtypeStruct((B, PEMB), jnp.float32),
    scratch_types=[
        pltpu.VMEM((NCH, CH), jnp.int32),
        pltpu.VMEM((NBUF, CH, PEMB), jnp.float32),
        [pltpu.SemaphoreType.DMA] * NBUF,
        [pltpu.SemaphoreType.DMA] * NBUF,
    ],
    compiler_params=pltpu.CompilerParams(use_tc_tiling_on_sc=True),
)
def _gather_kernel(tok_hbm, table_hbm, out_hbm, idx_v, rows_v, gsems, wsems):
    wid = lax.axis_index("s") * _NC + lax.axis_index("c")
    base = wid * B_PER_W
    pltpu.sync_copy(tok_hbm.at[wid], idx_v)

    def gather(j, b):
        pltpu.async_copy(table_hbm.at[idx_v.at[j]], rows_v.at[b], gsems[b])

    def write(j, b):
        pltpu.async_copy(rows_v.at[b], out_hbm.at[pl.ds(base + j * CH, CH)],
                         wsems[b])

    def wait_gather(j, b):
        pltpu.make_async_copy(table_hbm.at[idx_v.at[b]], rows_v.at[b],
                              gsems[b]).wait()

    def wait_write(j, b):
        pltpu.make_async_copy(rows_v.at[b],
                              out_hbm.at[pl.ds(base + j * CH, CH)],
                              wsems[b]).wait()

    for b in range(NBUF):
        gather(b, b)

    def group(g, carry):
        for b in range(NBUF):
            j = g * NBUF + b
            wait_gather(j, b)
            write(j, b)
        for b in range(NBUF):
            j = g * NBUF + b
            wait_write(j, b)
            gather(j + NBUF, b)
        return carry

    lax.fori_loop(0, NCH // NBUF - 1, group, 0)

    for b in range(NBUF):
        j = NCH - NBUF + b
        wait_gather(j, b)
        write(j, b)
    for b in range(NBUF):
        j = NCH - NBUF + b
        wait_write(j, b)


VOCAB = 1000000
NFULL = VOCAB // CH
TAIL = VOCAB - NFULL * CH


@functools.partial(
    pl.kernel,
    mesh=_MESH,
    out_type=jax.ShapeDtypeStruct((VOCAB, PEMB), jnp.float32),
    scratch_types=[
        pltpu.VMEM((2, EMB, CH), jnp.float32),
        pltpu.VMEM((2, CH, PEMB), jnp.float32),
        [pltpu.SemaphoreType.DMA] * 2,
        [pltpu.SemaphoreType.DMA] * 2,
    ],
    compiler_params=pltpu.CompilerParams(use_tc_tiling_on_sc=True,
                                         needs_layout_passes=False),
)
def _fmt_kernel(tabt_hbm, tail_hbm, out_hbm, in_v, out_v, isems, osems):
    wid = lax.axis_index("s") * _NC + lax.axis_index("c")

    lane = lax.iota(jnp.int32, 16)
    NK = NFULL // NW

    def load(k, b):
        pltpu.async_copy(tabt_hbm.at[:, pl.ds((wid + k * NW) * CH, CH)],
                         in_v.at[b], isems[b])

    def wait_load(k, b):
        pltpu.make_async_copy(tabt_hbm.at[:, pl.ds((wid + k * NW) * CH, CH)],
                              in_v.at[b], isems[b]).wait()

    def store(k, b):
        pltpu.async_copy(out_v.at[b], out_hbm.at[pl.ds((wid + k * NW) * CH, CH)],
                         osems[b])

    def wait_store(k, b):
        pltpu.make_async_copy(out_v.at[b],
                              out_hbm.at[pl.ds((wid + k * NW) * CH, CH)],
                              osems[b]).wait()

    lanes = [lane + 16 * kk for kk in range(EMB // 16)]
    zero16 = jnp.full((16,), 0, jnp.int32)

    def transpose_block(b):
        def row4(r, carry):
            for dv in range(4):
                v = r * 4 + dv
                col = zero16 + v
                for kk in range(EMB // 16):
                    g = plsc.load_gather(in_v.at[b], [lanes[kk], col])
                    out_v[b, v, pl.ds(16 * kk, 16)] = g
            return carry
        lax.fori_loop(0, CH // 4, row4, 0)

    load(0, 0)
    load(1, 1)

    def group(g, carry):
        for b in range(2):
            k = 2 * g + b
            wait_load(k, b)
            transpose_block(b)
            store(k, b)
        for b in range(2):
            k = 2 * g + b
            wait_store(k, b)

            @pl.when(k + 2 < NK)
            def _():
                load(k + 2, b)

        return carry

    lax.fori_loop(0, NK // 2, group, 0)

    @pl.when(wid < NFULL - NK * NW)
    def _():
        pltpu.sync_copy(tabt_hbm.at[:, pl.ds((wid + NK * NW) * CH, CH)],
                        in_v.at[0])
        transpose_block(0)
        pltpu.sync_copy(out_v.at[0],
                        out_hbm.at[pl.ds((wid + NK * NW) * CH, CH)])

    @pl.when(wid == NW - 1)
    def _():
        pltpu.sync_copy(tail_hbm, out_v.at[1, pl.ds(0, TAIL)])
        pltpu.sync_copy(out_v.at[1, pl.ds(0, TAIL)],
                        out_hbm.at[pl.ds(NFULL * CH, TAIL)])


def kernel(tokens, table):
    tok = tokens.reshape(NW, NCH, CH)
    tail = jnp.pad(table[NFULL * CH:], ((0, 0), (0, PEMB - EMB)))
    tab = _fmt_kernel(table.T, tail)
    out = _gather_kernel(tok, tab)
    return out[:, :EMB].reshape(B_TOK, T_TOK, EMB)


# speedup vs baseline: 2.0733x; 2.0690x over previous
import functools

import jax
import jax.numpy as jnp
from jax import lax
from jax.experimental import pallas as pl
from jax.experimental.pallas import tpu as pltpu
from jax.experimental.pallas import tpu_sc as plsc

B_TOK = 4096
T_TOK = 200
EMB = 64
PEMB = 128
NW = 32
B = B_TOK * T_TOK
B_PER_W = B // NW
CH = 128
NCH = B_PER_W // CH

_NC = 2
NBUF = 4
_MESH = plsc.VectorSubcoreMesh(core_axis_name="c", subcore_axis_name="s")

VOCAB = 1000000


@functools.partial(
    pl.kernel,
    mesh=_MESH,
    out_type=jax.ShapeDtypeStruct((B, PEMB), jnp.float32),
    scratch_types=[
        pltpu.VMEM((NCH, CH), jnp.int32),
        pltpu.VMEM((NBUF, CH, PEMB), jnp.float32),
        [pltpu.SemaphoreType.DMA] * NBUF,
        [pltpu.SemaphoreType.DMA] * NBUF,
    ],
    compiler_params=pltpu.CompilerParams(use_tc_tiling_on_sc=True),
)
def _gather_kernel(tok_hbm, table_hbm, out_hbm, idx_v, rows_v, gsems, wsems):
    wid = lax.axis_index("s") * _NC + lax.axis_index("c")
    base = wid * B_PER_W
    pltpu.sync_copy(tok_hbm.at[wid], idx_v)

    def gather(j, b):
        pltpu.async_copy(table_hbm.at[idx_v.at[j]], rows_v.at[b], gsems[b])

    def write(j, b):
        pltpu.async_copy(rows_v.at[b], out_hbm.at[pl.ds(base + j * CH, CH)],
                         wsems[b])

    def wait_gather(j, b):
        pltpu.make_async_copy(table_hbm.at[idx_v.at[b]], rows_v.at[b],
                              gsems[b]).wait()

    def wait_write(j, b):
        pltpu.make_async_copy(rows_v.at[b],
                              out_hbm.at[pl.ds(base + j * CH, CH)],
                              wsems[b]).wait()

    for b in range(NBUF):
        gather(b, b)

    def group(g, carry):
        for b in range(NBUF):
            j = g * NBUF + b
            wait_gather(j, b)
            write(j, b)
        for b in range(NBUF):
            j = g * NBUF + b
            wait_write(j, b)
            gather(j + NBUF, b)
        return carry

    lax.fori_loop(0, NCH // NBUF - 1, group, 0)

    for b in range(NBUF):
        j = NCH - NBUF + b
        wait_gather(j, b)
        write(j, b)
    for b in range(NBUF):
        j = NCH - NBUF + b
        wait_write(j, b)


def kernel(tokens, table):
    tok = tokens.reshape(NW, NCH, CH)
    tab = jnp.pad(table, ((0, 0), (0, PEMB - EMB)))
    out = _gather_kernel(tok, tab)
    return out[:, :EMB].reshape(B_TOK, T_TOK, EMB)
